# Initial kernel scaffold; baseline (speedup 1.0000x reference)
import jax
import jax.numpy as jnp
from jax.experimental import pallas as pl


def kernel(graph_features, W1, b1, W2, b2, Wp, bp):
    raise NotImplementedError("write your pallas kernel here")


# speedup vs baseline: 1.7844x; 1.7844x over previous
import functools

import jax
import jax.numpy as jnp
from jax.experimental import pallas as pl

_B, _S, _D = 4, 2048, 2048
_NUM_NODES = 32
_NODE_DIM = _D // _NUM_NODES
_K = 8
_HID = _NODE_DIM // 2
_T = 512


def _fused_kernel(x_ref, w1_ref, b1_ref, w2_ref, wp_ref, bp_ref, out_ref):
    x3 = x_ref[...]
    t = x3.shape[0]
    x2 = x3.reshape(t * _NUM_NODES, _NODE_DIM)
    h = jnp.dot(
        x2.astype(jnp.bfloat16),
        w1_ref[...].astype(jnp.bfloat16),
        preferred_element_type=jnp.float32,
    )
    h = h + b1_ref[...]
    h = 0.5 * h * (1.0 + jax.lax.erf(h * 0.7071067811865476))
    s = jnp.sum(
        h.astype(jnp.bfloat16).astype(jnp.float32)
        * w2_ref[...].astype(jnp.bfloat16).astype(jnp.float32),
        axis=-1,
        keepdims=True,
    )
    s3 = s.reshape(t, _NUM_NODES, 1)

    iota = jax.lax.broadcasted_iota(jnp.int32, (t, _NUM_NODES, 1), 1)
    cur = s3
    parts = []
    for _ in range(_K):
        m = jnp.max(cur, axis=1, keepdims=True)
        idx_k = jnp.min(
            jnp.where(cur == m, iota, _NUM_NODES), axis=1, keepdims=True
        )
        onehot = iota == idx_k
        cur = jnp.where(onehot, -jnp.inf, cur)
        parts.append(jnp.sum(jnp.where(onehot, x3, 0.0), axis=1))

    pooled = jnp.concatenate(parts, axis=-1)
    out_ref[...] = (
        jnp.dot(pooled, wp_ref[...], preferred_element_type=jnp.float32)
        + bp_ref[...]
    )


@jax.jit
def kernel(graph_features, W1, b1, W2, b2, Wp, bp):
    del b2
    bsz, seqlen, dmodel = graph_features.shape
    n_tok = bsz * seqlen
    x = graph_features.reshape(n_tok, _NUM_NODES, _NODE_DIM)

    grid = (n_tok // _T,)
    out = pl.pallas_call(
        _fused_kernel,
        grid=grid,
        in_specs=[
            pl.BlockSpec((_T, _NUM_NODES, _NODE_DIM), lambda i: (i, 0, 0)),
            pl.BlockSpec((_NODE_DIM, _HID), lambda i: (0, 0)),
            pl.BlockSpec((1, _HID), lambda i: (0, 0)),
            pl.BlockSpec((1, _HID), lambda i: (0, 0)),
            pl.BlockSpec((_K * _NODE_DIM, _D), lambda i: (0, 0)),
            pl.BlockSpec((1, _D), lambda i: (0, 0)),
        ],
        out_specs=pl.BlockSpec((_T, _D), lambda i: (i, 0)),
        out_shape=jax.ShapeDtypeStruct((n_tok, _D), jnp.float32),
    )(
        x,
        W1,
        b1.reshape(1, _HID),
        W2.reshape(1, _HID),
        Wp,
        bp.reshape(1, _D),
    )
    return out.reshape(bsz, seqlen, dmodel)


# speedup vs baseline: 2.5099x; 1.4066x over previous
import functools

import jax
import jax.numpy as jnp
from jax.experimental import pallas as pl
from jax.experimental.pallas import tpu as pltpu

_B, _S, _D = 4, 2048, 2048
_NUM_NODES = 32
_NODE_DIM = _D // _NUM_NODES
_K = 8
_HID = _NODE_DIM // 2
_T = 512


def _fused_kernel(x_ref, w1_ref, b1_ref, w2_ref, wp_ref, bp_ref, out_ref):
    x3 = x_ref[...]
    t = x3.shape[0]
    x2 = x3.reshape(t * _NUM_NODES, _NODE_DIM)
    h = jnp.dot(
        x2.astype(jnp.bfloat16),
        w1_ref[...].astype(jnp.bfloat16),
        preferred_element_type=jnp.float32,
    )
    h = h + b1_ref[...]
    h = 0.5 * h * (1.0 + jax.lax.erf(h * 0.7071067811865476))
    s = jnp.sum(
        h.astype(jnp.bfloat16).astype(jnp.float32)
        * w2_ref[...].astype(jnp.bfloat16).astype(jnp.float32),
        axis=-1,
        keepdims=True,
    )
    s3 = s.reshape(t, _NUM_NODES, 1)

    iota = jax.lax.broadcasted_iota(jnp.int32, (t, _NUM_NODES, 1), 1)
    cur = s3
    idxs = []
    for _ in range(_K):
        m = jnp.max(cur, axis=1, keepdims=True)
        idx_k = jnp.min(
            jnp.where(cur == m, iota, _NUM_NODES), axis=1, keepdims=True
        )
        cur = jnp.where(iota == idx_k, -jnp.inf, cur)
        idxs.append(idx_k)

    idx = jnp.broadcast_to(
        jnp.concatenate(idxs, axis=1), (t, _K, _NODE_DIM)
    )
    gathered = jnp.zeros((t, _K, _NODE_DIM), jnp.float32)
    for g in range(_NUM_NODES // 8):
        idx_g = jnp.clip(idx - 8 * g, 0, 7)
        gath_g = jnp.take_along_axis(x3[:, 8 * g : 8 * (g + 1), :], idx_g, axis=1)
        gathered = jnp.where((idx // 8) == g, gath_g, gathered)
    pooled = gathered.reshape(t, _K * _NODE_DIM)
    out_ref[...] = (
        jnp.dot(pooled, wp_ref[...], preferred_element_type=jnp.float32)
        + bp_ref[...]
    )


@jax.jit
def kernel(graph_features, W1, b1, W2, b2, Wp, bp):
    del b2
    bsz, seqlen, dmodel = graph_features.shape
    n_tok = bsz * seqlen
    x = graph_features.reshape(n_tok, _NUM_NODES, _NODE_DIM)

    grid = (n_tok // _T,)
    out = pl.pallas_call(
        _fused_kernel,
        grid=grid,
        in_specs=[
            pl.BlockSpec((_T, _NUM_NODES, _NODE_DIM), lambda i: (i, 0, 0)),
            pl.BlockSpec((_NODE_DIM, _HID), lambda i: (0, 0)),
            pl.BlockSpec((1, _HID), lambda i: (0, 0)),
            pl.BlockSpec((1, _HID), lambda i: (0, 0)),
            pl.BlockSpec((_K * _NODE_DIM, _D), lambda i: (0, 0)),
            pl.BlockSpec((1, _D), lambda i: (0, 0)),
        ],
        out_specs=pl.BlockSpec((_T, _D), lambda i: (i, 0)),
        out_shape=jax.ShapeDtypeStruct((n_tok, _D), jnp.float32),
        compiler_params=pltpu.CompilerParams(
            dimension_semantics=("parallel",)
        ),
    )(
        x,
        W1,
        b1.reshape(1, _HID),
        W2.reshape(1, _HID),
        Wp,
        bp.reshape(1, _D),
    )
    return out.reshape(bsz, seqlen, dmodel)


# speedup vs baseline: 4.9218x; 1.9609x over previous
import jax
import jax.numpy as jnp
from jax import lax
from jax.experimental import pallas as pl
from jax.experimental.pallas import tpu as pltpu

_B, _S, _D = 4, 2048, 2048
_NUM_NODES = 32
_NODE_DIM = _D // _NUM_NODES
_K = 8
_HID = _NODE_DIM // 2
_T = 512


def _fused_kernel(x2_ref, x3_ref, w1_ref, b1_ref, w2_ref, wp_ref, bp_ref, out_ref):
    x2 = x2_ref[...]
    t = x2.shape[0]
    h = jnp.dot(
        x2.astype(jnp.bfloat16),
        w1_ref[...],
        preferred_element_type=jnp.float32,
    )
    h = h + b1_ref[...]
    h = 0.5 * h * (1.0 + lax.erf(h * 0.7071067811865476))
    s = jnp.dot(
        h.astype(jnp.bfloat16),
        w2_ref[...],
        preferred_element_type=jnp.float32,
    )

    iota = lax.broadcasted_iota(jnp.int32, (t, _NUM_NODES), 1)
    cur = s
    idxs = []
    for _ in range(_K):
        m = jnp.max(cur, axis=1, keepdims=True)
        idx_k = jnp.min(
            jnp.where(cur == m, iota, _NUM_NODES), axis=1, keepdims=True
        )
        cur = jnp.where(iota == idx_k, -jnp.inf, cur)
        idxs.append(idx_k)

    idx = jnp.concatenate(idxs, axis=1)
    idx3 = jnp.broadcast_to(
        idx.reshape(t, _K, 1), (t, _K, _NODE_DIM)
    )

    x3 = x3_ref[...]
    gathered = jnp.zeros((t, _K, _NODE_DIM), jnp.float32)
    for g in range(_NUM_NODES // 8):
        idx_g = jnp.clip(idx3 - 8 * g, 0, 7)
        gath_g = jnp.take_along_axis(x3[:, 8 * g : 8 * (g + 1), :], idx_g, axis=1)
        gathered = jnp.where((idx3 // 8) == g, gath_g, gathered)

    pooled = gathered.reshape(t, _K * _NODE_DIM)
    out_ref[...] = (
        jnp.dot(pooled, wp_ref[...], preferred_element_type=jnp.float32)
        + bp_ref[...]
    )


@jax.jit
def kernel(graph_features, W1, b1, W2, b2, Wp, bp):
    del b2
    bsz, seqlen, dmodel = graph_features.shape
    n_tok = bsz * seqlen
    x2 = graph_features.reshape(n_tok, _D)
    x3 = graph_features.reshape(n_tok, _NUM_NODES, _NODE_DIM)

    eye = jnp.eye(_NUM_NODES, dtype=jnp.float32)
    w1bd = (eye[:, None, :, None] * W1[None, :, None, :]).reshape(
        _D, _NUM_NODES * _HID
    ).astype(jnp.bfloat16)
    w2bd = (eye[:, None, :] * W2[None, :, 0, None]).reshape(
        _NUM_NODES * _HID, _NUM_NODES
    ).astype(jnp.bfloat16)
    b1bd = jnp.tile(b1, _NUM_NODES).reshape(1, _NUM_NODES * _HID)

    grid = (n_tok // _T,)
    out = pl.pallas_call(
        _fused_kernel,
        grid=grid,
        in_specs=[
            pl.BlockSpec((_T, _D), lambda i: (i, 0)),
            pl.BlockSpec((_T, _NUM_NODES, _NODE_DIM), lambda i: (i, 0, 0)),
            pl.BlockSpec((_D, _NUM_NODES * _HID), lambda i: (0, 0)),
            pl.BlockSpec((1, _NUM_NODES * _HID), lambda i: (0, 0)),
            pl.BlockSpec((_NUM_NODES * _HID, _NUM_NODES), lambda i: (0, 0)),
            pl.BlockSpec((_K * _NODE_DIM, _D), lambda i: (0, 0)),
            pl.BlockSpec((1, _D), lambda i: (0, 0)),
        ],
        out_specs=pl.BlockSpec((_T, _D), lambda i: (i, 0)),
        out_shape=jax.ShapeDtypeStruct((n_tok, _D), jnp.float32),
        compiler_params=pltpu.CompilerParams(
            dimension_semantics=("parallel",)
        ),
    )(x2, x3, w1bd, b1bd, w2bd, Wp, bp.reshape(1, _D))
    return out.reshape(bsz, seqlen, dmodel)


# speedup vs baseline: 5.4540x; 1.1081x over previous
import functools

import jax
import jax.numpy as jnp
from jax import lax
from jax.experimental import pallas as pl
from jax.experimental.pallas import tpu as pltpu
from jax.experimental.pallas import tpu_sc as plsc

_B, _S, _D = 4, 2048, 2048
_NUM_NODES = 32
_NODE_DIM = _D // _NUM_NODES
_K = 8
_HID = _NODE_DIM // 2
_T = 512

_N_TOK = _B * _S
_ROWS = _N_TOK * _K
_PAIR_W = 2 * _NODE_DIM
_NC, _NS = 2, 16
_NW = _NC * _NS
_RPW = _ROWS // _NW
_CH = 128
_NCH = _RPW // _CH


def _score_kernel(x2_ref, w1_ref, b1_ref, w2_ref, idx_ref, pair_ref):
    x2 = x2_ref[...]
    t = x2.shape[0]
    h = jnp.dot(
        x2.astype(jnp.bfloat16),
        w1_ref[...],
        preferred_element_type=jnp.float32,
    )
    h = h + b1_ref[...]
    h = 0.5 * h * (1.0 + lax.erf(h * 0.7071067811865476))
    s = jnp.dot(
        h.astype(jnp.bfloat16),
        w2_ref[...],
        preferred_element_type=jnp.float32,
    )

    iota = lax.broadcasted_iota(jnp.int32, (t, _NUM_NODES), 1)
    cur = s
    idxs = []
    for _ in range(_K):
        m = jnp.max(cur, axis=1, keepdims=True)
        idx_k = jnp.min(
            jnp.where(cur == m, iota, _NUM_NODES), axis=1, keepdims=True
        )
        cur = jnp.where(iota == idx_k, -jnp.inf, cur)
        idxs.append(idx_k)

    idx = jnp.concatenate(idxs, axis=1)
    tok = pl.program_id(0) * t + lax.broadcasted_iota(jnp.int32, (t, _K), 0)
    flat = tok * _NUM_NODES + idx
    idx_ref[...] = flat
    pair_ref[...] = flat >> 1


def _sc_gather_kernel(table_hbm, pair_hbm, out_hbm, idx_v, rows_v, sem):
    wid = lax.axis_index("s") * _NC + lax.axis_index("c")
    base = wid * _RPW

    def body(i, carry):
        off = base + i * _CH
        pltpu.sync_copy(pair_hbm.at[pl.ds(off, _CH)], idx_v)
        pltpu.async_copy(table_hbm.at[idx_v], rows_v, sem).wait()
        pltpu.sync_copy(rows_v, out_hbm.at[pl.ds(off, _CH)])
        return carry

    lax.fori_loop(0, _NCH, body, 0)


def _proj_kernel(raw_ref, idx_ref, wp_ref, bp_ref, out_ref):
    raw = raw_ref[...]
    t = raw.shape[0]
    parity = idx_ref[...] & 1
    par3 = parity.reshape(t, _K, 1)
    sel = jnp.where(par3 == 1, raw[:, :, _NODE_DIM:], raw[:, :, :_NODE_DIM])
    pooled = sel.reshape(t, _K * _NODE_DIM)
    out_ref[...] = (
        jnp.dot(pooled, wp_ref[...], preferred_element_type=jnp.float32)
        + bp_ref[...]
    )


@jax.jit
def kernel(graph_features, W1, b1, W2, b2, Wp, bp):
    del b2
    bsz, seqlen, dmodel = graph_features.shape
    n_tok = bsz * seqlen
    x2 = graph_features.reshape(n_tok, _D)

    eye = jnp.eye(_NUM_NODES, dtype=jnp.float32)
    w1bd = (eye[:, None, :, None] * W1[None, :, None, :]).reshape(
        _D, _NUM_NODES * _HID
    ).astype(jnp.bfloat16)
    w2bd = (eye[:, None, :] * W2[None, :, 0, None]).reshape(
        _NUM_NODES * _HID, _NUM_NODES
    ).astype(jnp.bfloat16)
    b1bd = jnp.tile(b1, _NUM_NODES).reshape(1, _NUM_NODES * _HID)

    grid = (n_tok // _T,)
    flat_idx, pair_idx = pl.pallas_call(
        _score_kernel,
        grid=grid,
        in_specs=[
            pl.BlockSpec((_T, _D), lambda i: (i, 0)),
            pl.BlockSpec((_D, _NUM_NODES * _HID), lambda i: (0, 0)),
            pl.BlockSpec((1, _NUM_NODES * _HID), lambda i: (0, 0)),
            pl.BlockSpec((_NUM_NODES * _HID, _NUM_NODES), lambda i: (0, 0)),
        ],
        out_specs=[
            pl.BlockSpec((_T, _K), lambda i: (i, 0)),
            pl.BlockSpec((_T, _K), lambda i: (i, 0)),
        ],
        out_shape=[
            jax.ShapeDtypeStruct((n_tok, _K), jnp.int32),
            jax.ShapeDtypeStruct((n_tok, _K), jnp.int32),
        ],
        compiler_params=pltpu.CompilerParams(
            dimension_semantics=("parallel",)
        ),
    )(x2, w1bd, b1bd, w2bd)

    table = graph_features.reshape(n_tok * _NUM_NODES // 2, _PAIR_W)
    gather = functools.partial(
        pl.kernel,
        mesh=plsc.VectorSubcoreMesh(core_axis_name="c", subcore_axis_name="s"),
        out_type=jax.ShapeDtypeStruct((_ROWS, _PAIR_W), jnp.float32),
        scratch_types=[
            pltpu.VMEM((_CH,), jnp.int32),
            pltpu.VMEM((_CH, _PAIR_W), jnp.float32),
            pltpu.SemaphoreType.DMA,
        ],
    )(_sc_gather_kernel)
    raw = gather(table, pair_idx.reshape(_ROWS))

    raw3 = raw.reshape(n_tok, _K, _PAIR_W)
    out = pl.pallas_call(
        _proj_kernel,
        grid=grid,
        in_specs=[
            pl.BlockSpec((_T, _K, _PAIR_W), lambda i: (i, 0, 0)),
            pl.BlockSpec((_T, _K), lambda i: (i, 0)),
            pl.BlockSpec((_K * _NODE_DIM, _D), lambda i: (0, 0)),
            pl.BlockSpec((1, _D), lambda i: (0, 0)),
        ],
        out_specs=pl.BlockSpec((_T, _D), lambda i: (i, 0)),
        out_shape=jax.ShapeDtypeStruct((n_tok, _D), jnp.float32),
        compiler_params=pltpu.CompilerParams(
            dimension_semantics=("parallel",)
        ),
    )(raw3, flat_idx, Wp, bp.reshape(1, _D))
    return out.reshape(bsz, seqlen, dmodel)
